# baseline (speedup 1.0000x reference)
import functools

import jax
import jax.numpy as jnp
from jax import lax
from jax.experimental import pallas as pl
from jax.experimental.pallas import tpu as pltpu
from jax.experimental.pallas import tpu_sc as plsc

F32 = jnp.float32
I32 = jnp.int32

H = 384
B = 256
BLK_N = 256
CSR_CHUNK = 2048
SC_NW = 32
SC_CH = 128
SC_ALIGN = SC_NW * SC_CH
_SENT = 2 ** 30


def _cdiv(a, b):
    return (a + b - 1) // b


def _blk_rows(rows):
    for c in (512, 400, 256, 128, 64, 32, 16, 8):
        if rows % c == 0:
            return c
    raise ValueError(f"bad row count {rows}")


SC_NB = 2


def _sc_gather_multi(specs):
    D = specs[0][0].shape[1]
    dt = specs[0][0].dtype
    nt = len(specs)
    Es = [idx.shape[0] for _, idx in specs]
    for t, i in specs:
        assert t.shape[1] == D and t.dtype == dt
        assert i.shape[0] % SC_ALIGN == 0
    mesh = plsc.VectorSubcoreMesh(core_axis_name="c", subcore_axis_name="s")

    @functools.partial(
        pl.kernel, mesh=mesh,
        compiler_params=pltpu.CompilerParams(
            use_tc_tiling_on_sc=(D % 128 == 0)),
        out_type=[jax.ShapeDtypeStruct((E, D), dt) for E in Es],
        scratch_types=(
            [pltpu.VMEM((SC_CH,), I32) for _ in range(SC_NB)]
            + [pltpu.VMEM((SC_CH, D), dt) for _ in range(SC_NB)]
            + [pltpu.SemaphoreType.DMA] * (3 * SC_NB)
        ),
    )
    def k(*refs):
        tab_refs = refs[:nt]
        idx_refs = refs[nt:2 * nt]
        out_refs = refs[2 * nt:3 * nt]
        sc = refs[3 * nt:]
        idx_bufs = sc[:SC_NB]
        row_bufs = sc[SC_NB:2 * SC_NB]
        sem_i = sc[2 * SC_NB:3 * SC_NB]
        sem_g = sc[3 * SC_NB:4 * SC_NB]
        sem_o = sc[4 * SC_NB:5 * SC_NB]
        wid = lax.axis_index("s") * 2 + lax.axis_index("c")

        def run_batch(t, gbase, nb):
            base = [gbase + b * SC_CH for b in range(nb)]
            hi = [pltpu.async_copy(
                idx_refs[t].at[pl.ds(base[b], SC_CH)], idx_bufs[b], sem_i[b])
                for b in range(nb)]
            for b in range(nb):
                hi[b].wait()
            hg = [pltpu.async_copy(
                tab_refs[t].at[idx_bufs[b]], row_bufs[b], sem_g[b])
                for b in range(nb)]
            for b in range(nb):
                hg[b].wait()
            ho = [pltpu.async_copy(
                row_bufs[b], out_refs[t].at[pl.ds(base[b], SC_CH)], sem_o[b])
                for b in range(nb)]
            for b in range(nb):
                ho[b].wait()

        for t in range(nt):
            per_w = Es[t] // SC_NW
            n_ch = per_w // SC_CH
            n_grp, rem = n_ch // SC_NB, n_ch % SC_NB

            if n_grp:
                def grp_body(g, carry, t=t, per_w=per_w):
                    run_batch(t, wid * per_w + g * (SC_NB * SC_CH), SC_NB)
                    return carry
                lax.fori_loop(0, n_grp, grp_body, 0)
            if rem:
                run_batch(t, wid * per_w + n_grp * (SC_NB * SC_CH), rem)

    outs = k(*[t for t, _ in specs], *[i for _, i in specs])
    return outs if isinstance(outs, (list, tuple)) else [outs]


def _sc_gather(table, idx):
    return _sc_gather_multi([(table, idx)])[0]


def _csr_segsum(vals, seg2d, starts, nblk, D, aux16=None, scale_by_aux=False,
                also_unscaled=False, with_cnt=False, chunk=CSR_CHUNK):
    rows = vals.shape[0]
    chunk = min(chunk, rows)
    assert rows % 8 == 0
    n_out = 1 + (aux16 is not None) + also_unscaled + with_cnt

    def body(*refs):
        starts_ref, vals_ref = refs[0], refs[1]
        k = 2
        aux_ref = None
        if aux16 is not None:
            aux_ref = refs[k]
            k += 1
        seg_ref = refs[k]
        k += 1
        outs = refs[k:k + n_out]
        k += n_out
        vchunk, schunk, sem_v, sem_s = refs[k:k + 4]
        k += 4
        achunk = sem_a = None
        if aux16 is not None:
            achunk, sem_a = refs[k:k + 2]

        i = pl.program_id(0)
        start = starts_ref[i]
        end = starts_ref[i + 1]
        for o_ref in outs:
            o_ref[...] = jnp.zeros_like(o_ref)
        start8 = (start // 8) * 8
        nch = (end - start8 + chunk - 1) // chunk

        def chunk_body(c, carry):
            cb = start8 + c * chunk
            o = jnp.minimum(cb, rows - chunk)
            cps = [pltpu.make_async_copy(
                vals_ref.at[pl.ds(o, chunk)], vchunk, sem_v),
                pltpu.make_async_copy(
                seg_ref.at[pl.ds(o, chunk)], schunk, sem_s)]
            if aux_ref is not None:
                cps.append(pltpu.make_async_copy(
                    aux_ref.at[pl.ds(o, chunk)], achunk, sem_a))
            for cp in cps:
                cp.start()
            for cp in cps:
                cp.wait()
            g = o + lax.broadcasted_iota(I32, (chunk, 1), 0)
            valid = ((g >= jnp.maximum(start, cb))
                     & (g < jnp.minimum(end, cb + chunk)))
            local = schunk[...] - i * BLK_N
            oh = ((local == lax.broadcasted_iota(I32, (chunk, BLK_N), 1))
                  & valid).astype(F32)

            def acc(o_ref, rhs):
                o_ref[...] += lax.dot_general(
                    oh, rhs, (((0,), (0,)), ((), ())),
                    precision=lax.Precision.HIGHEST,
                    preferred_element_type=F32)

            oi = 0
            v = vchunk[...]
            if scale_by_aux:
                acc(outs[oi], achunk[...][:, :1] * v)
            else:
                acc(outs[oi], v)
            oi += 1
            if aux16 is not None:
                acc(outs[oi], achunk[...])
                oi += 1
            if also_unscaled:
                acc(outs[oi], v)
                oi += 1
            if with_cnt:
                acc(outs[oi], jnp.ones((chunk, 16), F32))
            return carry

        lax.fori_loop(0, nch, chunk_body, 0)

    in_specs = [pl.BlockSpec(memory_space=pltpu.SMEM),
                pl.BlockSpec(memory_space=pl.ANY)]
    args = [starts, vals]
    if aux16 is not None:
        in_specs.append(pl.BlockSpec(memory_space=pl.ANY))
        args.append(aux16)
    in_specs.append(pl.BlockSpec(memory_space=pl.ANY))
    args.append(seg2d)

    out_D = ([D] + ([16] if aux16 is not None else [])
             + ([D] if also_unscaled else []) + ([16] if with_cnt else []))
    scratch = [pltpu.VMEM((chunk, D), F32),
               pltpu.VMEM((chunk, 1), I32),
               pltpu.SemaphoreType.DMA,
               pltpu.SemaphoreType.DMA]
    if aux16 is not None:
        scratch += [pltpu.VMEM((chunk, 16), F32), pltpu.SemaphoreType.DMA]

    res = pl.pallas_call(
        body,
        grid=(nblk,),
        in_specs=in_specs,
        out_specs=[pl.BlockSpec((BLK_N, d), lambda i: (i, 0)) for d in out_D],
        out_shape=[jax.ShapeDtypeStruct((nblk * BLK_N, d), F32) for d in out_D],
        scratch_shapes=scratch,
    )(*args)
    return res[0] if n_out == 1 else res


_BF16_GATHER = False


def _dense(parts, b1, mid, W2, b2, out_act, res, ln, divs=None,
           bf16_copy=False):
    dup_out = bf16_copy and not _BF16_GATHER
    bf16_copy = bf16_copy and _BF16_GATHER
    rows = parts[0][0].shape[0]
    blk = _blk_rows(rows)
    grid = rows // blk
    H1 = parts[0][1].shape[1]
    Hout = W2.shape[1] if W2 is not None else H1
    divs = divs or [None] * len(parts)

    arrs = [a for a, _ in parts]
    Ws = [jnp.asarray(w) for _, w in parts]
    n = len(parts)
    div_idx = [j for j, d in enumerate(divs) if d is not None]
    div_arrs = [divs[j][0] for j in div_idx]
    div_modes = [divs[j][1] for j in div_idx]

    tail = [b1.reshape(1, -1)]
    if W2 is not None:
        tail += [W2, b2.reshape(1, -1)]
    if ln is not None:
        tail += [ln["g"].reshape(1, -1), ln["b"].reshape(1, -1)]
    if res is not None:
        tail += [res]

    def body(*refs):
        a_refs = refs[:n]
        w_refs = refs[n:2 * n]
        pos = 2 * n
        d_refs = refs[pos:pos + len(div_arrs)]
        pos += len(div_arrs)
        b1_ref = refs[pos]
        pos += 1
        if W2 is not None:
            W2_ref, b2_ref = refs[pos], refs[pos + 1]
            pos += 2
        if ln is not None:
            g_ref, be_ref = refs[pos], refs[pos + 1]
            pos += 2
        if res is not None:
            r_ref = refs[pos]
            pos += 1
        out_ref = refs[pos:pos + 2] if bf16_copy else refs[pos]

        h = None
        di = 0
        for j in range(n):
            a = a_refs[j][...]
            if j in div_idx:
                dv = d_refs[di][...][:, :1]
                if div_modes[di] == "cnt":
                    a = a / jnp.maximum(dv, 1.0)
                else:
                    a = a / (dv + 1e-8)
                di += 1
            t = jnp.dot(a, w_refs[j][...], preferred_element_type=F32)
            h = t if h is None else h + t
        h = h + b1_ref[...]
        if mid == "silu":
            h = h * jax.nn.sigmoid(h)
        elif mid == "tanh":
            h = jnp.tanh(h)
        if W2 is not None:
            h = jnp.dot(h, W2_ref[...], preferred_element_type=F32) + b2_ref[...]
        if out_act == "silu":
            h = h * jax.nn.sigmoid(h)
        if res is not None:
            h = h + r_ref[...]
        if ln is not None:
            m = jnp.mean(h, axis=1, keepdims=True)
            v = jnp.mean((h - m) ** 2, axis=1, keepdims=True)
            h = (h - m) * lax.rsqrt(v + 1e-5) * g_ref[...] + be_ref[...]
        if bf16_copy:
            out_ref, bf_ref = out_ref
            bf_ref[...] = h.astype(jnp.bfloat16)
        out_ref[...] = h

    in_specs = (
        [pl.BlockSpec((blk, a.shape[1]), lambda i: (i, 0)) for a in arrs]
        + [pl.BlockSpec(w.shape, lambda i: (0, 0)) for w in Ws]
        + [pl.BlockSpec((blk, d.shape[1]), lambda i: (i, 0)) for d in div_arrs]
    )
    for t in tail[:-1] if res is not None else tail:
        in_specs.append(pl.BlockSpec(t.shape, lambda i: (0, 0)))
    if res is not None:
        in_specs.append(pl.BlockSpec((blk, Hout), lambda i: (i, 0)))

    res_ = pl.pallas_call(
        body,
        grid=(grid,),
        in_specs=in_specs,
        out_specs=([pl.BlockSpec((blk, Hout), lambda i: (i, 0))] * 2
                   if bf16_copy else
                   pl.BlockSpec((blk, Hout), lambda i: (i, 0))),
        out_shape=([jax.ShapeDtypeStruct((rows, Hout), F32),
                    jax.ShapeDtypeStruct((rows, Hout), jnp.bfloat16)]
                   if bf16_copy else
                   jax.ShapeDtypeStruct((rows, Hout), F32)),
    )(*arrs, *Ws, *div_arrs, *tail)
    if dup_out:
        return res_, res_
    return res_


def _gec_edge(ea_s, rbf_s, xg, q):
    rows = ea_s.shape[0]
    blk = 512
    edp = ea_s.shape[1]
    W1a = _zero_pad_W(q["ee1"]["W"][:-16], edp)
    W1b = q["ee1"]["W"][-16:]
    Wm = q["mm1"]["W"]

    def body(ea_ref, rb_ref, xg_ref, w1a, w1b, b1, w2, b2, wma, wmb, bm,
             wm2, bm2, out):
        h = (jnp.dot(ea_ref[...], w1a[...], preferred_element_type=F32)
             + jnp.dot(rb_ref[...], w1b[...], preferred_element_type=F32)
             + b1[...])
        h = h * jax.nn.sigmoid(h)
        e = jnp.dot(h, w2[...], preferred_element_type=F32) + b2[...]
        m = (jnp.dot(xg_ref[...].astype(F32), wma[...],
                     preferred_element_type=F32)
             + jnp.dot(e, wmb[...], preferred_element_type=F32) + bm[...])
        m = m * jax.nn.sigmoid(m)
        out[...] = jnp.dot(m, wm2[...], preferred_element_type=F32) + bm2[...]

    args = [ea_s, rbf_s, xg, W1a, W1b, q["ee1"]["b"].reshape(1, -1),
            q["ee2"]["W"], q["ee2"]["b"].reshape(1, -1),
            Wm[:H], Wm[H:], q["mm1"]["b"].reshape(1, -1),
            q["mm2"]["W"], q["mm2"]["b"].reshape(1, -1)]
    in_specs = [pl.BlockSpec((blk, edp), lambda i: (i, 0)),
                pl.BlockSpec((blk, 16), lambda i: (i, 0)),
                pl.BlockSpec((blk, H), lambda i: (i, 0))]
    in_specs += [pl.BlockSpec(a.shape, lambda i: (0, 0)) for a in args[3:]]
    return pl.pallas_call(
        body, grid=(rows // blk,),
        in_specs=in_specs,
        out_specs=pl.BlockSpec((blk, H), lambda i: (i, 0)),
        out_shape=jax.ShapeDtypeStruct((rows, H), F32),
    )(*args)


def _rbf16(posr, posc):
    rows = posr.shape[0]
    blk = _blk_rows(rows)

    def body(pr, pc, out):
        diff = pr[...] - pc[...]
        d = jnp.sqrt(jnp.sum(diff * diff, axis=1, keepdims=True))
        c = lax.broadcasted_iota(I32, (1, 16), 1).astype(F32) * (10.0 / 15.0)
        out[...] = jnp.exp(-((d - c) ** 2) / (2.0 * 0.625 ** 2))

    return pl.pallas_call(
        body, grid=(rows // blk,),
        in_specs=[pl.BlockSpec((blk, 16), lambda i: (i, 0))] * 2,
        out_specs=pl.BlockSpec((blk, 16), lambda i: (i, 0)),
        out_shape=jax.ShapeDtypeStruct((rows, 16), F32),
    )(posr, posc)


def _score16(Qg, Kg, ceag, Web):
    rows = Qg.shape[0]
    blk = _blk_rows(rows)
    scale = float(H) ** -0.5

    def body(q, k, ce, w, out):
        s = jnp.sum(q[...] * k[...], axis=1, keepdims=True) * scale
        s = s + jnp.dot(ce[...], w[...], preferred_element_type=F32)
        out[...] = jnp.broadcast_to(s, (blk, 16))

    return pl.pallas_call(
        body, grid=(rows // blk,),
        in_specs=[
            pl.BlockSpec((blk, H), lambda i: (i, 0)),
            pl.BlockSpec((blk, H), lambda i: (i, 0)),
            pl.BlockSpec((blk, ceag.shape[1]), lambda i: (i, 0)),
            pl.BlockSpec(Web.shape, lambda i: (0, 0)),
        ],
        out_specs=pl.BlockSpec((blk, 16), lambda i: (i, 0)),
        out_shape=jax.ShapeDtypeStruct((rows, 16), F32),
    )(Qg, Kg, ceag, Web)


def _gmax(x):
    rows, C = x.shape
    blk = _blk_rows(rows)

    def body(x_ref, out):
        i = pl.program_id(0)
        m = jnp.max(x_ref[...]).reshape(1, 1)
        prev = jnp.where(i == 0, m, out[...])
        out[...] = jnp.maximum(prev, m)

    return pl.pallas_call(
        body, grid=(rows // blk,),
        in_specs=[pl.BlockSpec((blk, C), lambda i: (i, 0))],
        out_specs=pl.BlockSpec((1, 1), lambda i: (0, 0)),
        out_shape=jax.ShapeDtypeStruct((1, 1), F32),
    )(x)


def _sub_exp16(x, m):
    rows, C = x.shape
    blk = _blk_rows(rows)

    def body(x_ref, m_ref, out):
        v = jnp.exp(x_ref[...] - m_ref[...])
        out[...] = jnp.broadcast_to(v[:, :1], (blk, 16))

    return pl.pallas_call(
        body, grid=(rows // blk,),
        in_specs=[pl.BlockSpec((blk, C), lambda i: (i, 0)),
                  pl.BlockSpec((1, 1), lambda i: (0, 0))],
        out_specs=pl.BlockSpec((blk, 16), lambda i: (i, 0)),
        out_shape=jax.ShapeDtypeStruct((rows, 16), F32),
    )(x, m)


def _emul(e16, V):
    rows, D = V.shape
    blk = _blk_rows(rows)

    def body(e_ref, v_ref, out):
        out[...] = e_ref[...][:, :1] * v_ref[...]

    return pl.pallas_call(
        body, grid=(rows // blk,),
        in_specs=[pl.BlockSpec((blk, 16), lambda i: (i, 0)),
                  pl.BlockSpec((blk, D), lambda i: (i, 0))],
        out_specs=pl.BlockSpec((blk, D), lambda i: (i, 0)),
        out_shape=jax.ShapeDtypeStruct((rows, D), F32),
    )(e16, V)


def _bcast256(batch2d, table):
    rows = batch2d.shape[0]
    D = table.shape[1]
    blk = _blk_rows(rows)

    def body(b_ref, t_ref, out):
        oh = (b_ref[...] == lax.broadcasted_iota(I32, (blk, 256), 1)
              ).astype(F32)
        out[...] = lax.dot_general(
            oh, t_ref[...], (((1,), (0,)), ((), ())),
            precision=lax.Precision.HIGHEST, preferred_element_type=F32)

    return pl.pallas_call(
        body, grid=(rows // blk,),
        in_specs=[pl.BlockSpec((blk, 1), lambda i: (i, 0)),
                  pl.BlockSpec((256, D), lambda i: (0, 0))],
        out_specs=pl.BlockSpec((blk, D), lambda i: (i, 0)),
        out_shape=jax.ShapeDtypeStruct((rows, D), F32),
    )(batch2d, table)


def _pad_cols(x, width):
    t = width - x.shape[1]
    return jnp.pad(x, ((0, 0), (0, t))) if t else x


def _pad_idx(idx, mult):
    t = (-idx.shape[0]) % mult
    return jnp.pad(idx, (0, t)) if t else idx


def _seg2d(seg_sorted, rows):
    t = rows - seg_sorted.shape[0]
    s = (jnp.pad(seg_sorted, (0, t), constant_values=_SENT)
         if t else seg_sorted)
    return s.reshape(rows, 1)


def _starts(seg_sorted, nblk):
    bounds = jnp.arange(nblk + 1, dtype=I32) * BLK_N
    return jnp.searchsorted(seg_sorted, bounds).astype(I32)


def _zero_pad_W(Wpart, width):
    t = width - Wpart.shape[0]
    return jnp.pad(Wpart, ((0, t), (0, 0))) if t else Wpart


def _pack_i32(xb):
    if xb.dtype == F32:
        return xb
    r, c = xb.shape
    return lax.bitcast_convert_type(xb.reshape(r, c // 2, 2), I32)


def _unpack_bf16(xi):
    if xi.dtype == F32:
        return xi
    r, c = xi.shape
    return lax.bitcast_convert_type(xi, jnp.bfloat16).reshape(r, 2 * c)


def _gec_layer(q, x, x_bf, xg_idx, ea_s, rbf_s, seg2d, starts, nblk, N):
    xg = _unpack_bf16(_sc_gather(_pack_i32(x_bf), xg_idx))
    msg = _gec_edge(ea_s, rbf_s, xg, q)
    agg = _csr_segsum(msg, seg2d, starts, nblk, H, chunk=1024)[:N]
    Wu = q["um1"]["W"]
    return _dense(
        [(x, Wu[:H]), (agg, Wu[H:])],
        q["um1"]["b"], "silu", q["um2"]["W"], q["um2"]["b"], None, x,
        q["norm"], bf16_copy=True)


def _vn_layer(q, x, batch2d, starts_b, vn):
    sums, cnt = _csr_segsum(x, batch2d, starts_b, 1, H, with_cnt=True,
                            chunk=4096)
    Wt = q["to1"]["W"]
    vn_new = _dense(
        [(sums, Wt[:H]), (vn, Wt[H:])],
        q["to1"]["b"], "silu", q["to2"]["W"], q["to2"]["b"], None, vn,
        q["vn_norm"], divs=[(cnt, "cnt"), None])
    vng = _bcast256(batch2d, vn_new)
    Wf = q["fr1"]["W"]
    x_new = _dense(
        [(x, Wf[:H]), (vng, Wf[H:])],
        q["fr1"]["b"], "silu", q["fr2"]["W"], q["fr2"]["b"], None, x,
        q["node_norm"], bf16_copy=True)
    return x_new, vn_new


def _ca_side(x_tgt, Qg, Kg, Vg, cea_s, Web, seg2d, starts, nblk, Wout, bout,
             norm):
    N = x_tgt.shape[0]
    s16 = _score16(Qg, Kg, cea_s, Web)
    smax = _gmax(s16)
    e16 = _sub_exp16(s16, smax)
    upd_raw, den16 = _csr_segsum(Vg, seg2d, starts, nblk, H, aux16=e16,
                                 scale_by_aux=True, chunk=1024)
    return _dense([(upd_raw[:N], Wout)], bout, None, None, None, None, x_tgt,
                  norm, divs=[(den16[:N], "eps")], bf16_copy=True)


def _ca_apply(q, pro_x, lig_x, pi_sp, li_sp, pi_seg2d, pstarts, nblk_p,
              li_sp2, pi_sp2, li_seg2d, lstarts, nblk_l, cea_sp, cea_sl):
    Web = _zero_pad_W(q["eb"]["W"], 32)
    lin = lambda x, w: _dense([(x, w["W"])], w["b"], None, None, None, None,
                              None, None)
    q_pro, k_lig, v_lig = lin(pro_x, q["q_pro"]), lin(lig_x, q["k_lig"]), lin(lig_x, q["v_lig"])
    q_lig, k_pro, v_pro = lin(lig_x, q["q_lig"]), lin(pro_x, q["k_pro"]), lin(pro_x, q["v_pro"])
    QgP, KgP, VgP, QgL, KgL, VgL = _sc_gather_multi([
        (q_pro, pi_sp), (k_lig, li_sp), (v_lig, li_sp),
        (q_lig, li_sp2), (k_pro, pi_sp2), (v_pro, pi_sp2)])
    pro_new = _ca_side(pro_x, QgP, KgP, VgP, cea_sp, Web, pi_seg2d, pstarts,
                       nblk_p, q["out_pro"]["W"], q["out_pro"]["b"],
                       q["norm_pro"])
    lig_new = _ca_side(lig_x, QgL, KgL, VgL, cea_sl, Web, li_seg2d, lstarts,
                       nblk_l, q["out_lig"]["W"], q["out_lig"]["b"],
                       q["norm_lig"])
    return pro_new, lig_new


def _attn_pool(q, x, batch2d, starts_b):
    g = _dense([(x, q["g1"]["W"])], q["g1"]["b"], "tanh", q["g2"]["W"],
               q["g2"]["b"], None, None, None)
    gmax = _gmax(g)
    e16 = _sub_exp16(g, gmax)
    attn_raw, den16, sums, cnt = _csr_segsum(
        x, batch2d, starts_b, 1, H, aux16=e16, scale_by_aux=True,
        also_unscaled=True, with_cnt=True, chunk=4096)
    return attn_raw, den16, sums, cnt


def kernel(ligand_x, ligand_edge_index, ligand_edge_attr, ligand_pos,
           ligand_batch, protein_x, protein_edge_index, protein_edge_attr,
           protein_pos, protein_batch, cross_edge_index, cross_edge_attr,
           params):
    p = params
    N_LIG, N_PRO = ligand_x.shape[0], protein_x.shape[0]
    E_LIG = ligand_edge_index.shape[1]
    E_PRO = protein_edge_index.shape[1]
    E_CROSS = cross_edge_index.shape[1]

    def edge_prep(ei, E, N):
        row, col = ei[0], ei[1]
        perm = jnp.argsort(col)
        row_s, col_s = row[perm], col[perm]
        E_pad = E + ((-E) % SC_ALIGN)
        nblk = _cdiv(N, BLK_N)
        return (_pad_idx(perm, SC_ALIGN), _pad_idx(row_s, SC_ALIGN),
                _pad_idx(col_s, SC_ALIGN), _seg2d(col_s, E_pad),
                _starts(col_s, nblk), nblk)

    (lperm_p, lrow_sp, lcol_sp, lseg2d, lstarts, lnblk) = edge_prep(
        ligand_edge_index, E_LIG, N_LIG)
    (pperm_p, prow_sp, pcol_sp, pseg2d, pstarts, pnblk) = edge_prep(
        protein_edge_index, E_PRO, N_PRO)

    ci, cj = cross_edge_index[0], cross_edge_index[1]
    cpermP = jnp.argsort(ci)
    pi_sP, li_sP = ci[cpermP], cj[cpermP]
    cpermL = jnp.argsort(cj)
    pi_sL, li_sL = ci[cpermL], cj[cpermL]
    E_CR_P = E_CROSS + ((-E_CROSS) % SC_ALIGN)
    cnblk_p, cnblk_l = _cdiv(N_PRO, BLK_N), _cdiv(N_LIG, BLK_N)
    cp_seg2d, cp_starts = _seg2d(pi_sP, E_CR_P), _starts(pi_sP, cnblk_p)
    cl_seg2d, cl_starts = _seg2d(li_sL, E_CR_P), _starts(li_sL, cnblk_l)
    pi_sP_p, li_sP_p = _pad_idx(pi_sP, SC_ALIGN), _pad_idx(li_sP, SC_ALIGN)
    pi_sL_p, li_sL_p = _pad_idx(pi_sL, SC_ALIGN), _pad_idx(li_sL, SC_ALIGN)

    lbatch2d = ligand_batch.reshape(N_LIG, 1)
    pbatch2d = protein_batch.reshape(N_PRO, 1)
    lstarts_b = jnp.array([0, N_LIG], I32)
    pstarts_b = jnp.array([0, N_PRO], I32)

    lpos16 = _pad_cols(ligand_pos, 16)
    ppos16 = _pad_cols(protein_pos, 16)
    lposr, lposc, pposr, pposc = _sc_gather_multi([
        (lpos16, lrow_sp), (lpos16, lcol_sp),
        (ppos16, prow_sp), (ppos16, pcol_sp)])
    lrbf_s = _rbf16(lposr, lposc)
    prbf_s = _rbf16(pposr, pposc)
    cea32 = _pad_cols(cross_edge_attr, 32)
    lea_s, pea_s, cea_sP, cea_sL = _sc_gather_multi([
        (_pad_cols(ligand_edge_attr, 32), lperm_p),
        (_pad_cols(protein_edge_attr, 32), pperm_p),
        (cea32, _pad_idx(cpermP, SC_ALIGN)),
        (cea32, _pad_idx(cpermL, SC_ALIGN))])

    lig_x, lig_xb = _dense([(ligand_x, p["lig_in"]["W"])], p["lig_in"]["b"],
                           "silu", None, None, None, None, None,
                           bf16_copy=True)
    pro_x, pro_xb = _dense([(protein_x, p["pro_in"]["W"])], p["pro_in"]["b"],
                           "silu", None, None, None, None, None,
                           bf16_copy=True)
    lig_vn = jnp.tile(p["lig_vn_init"], (B, 1))
    pro_vn = jnp.tile(p["pro_vn_init"], (B, 1))

    for q in p["lig_convs_a"]:
        lig_x, lig_xb = _gec_layer(q, lig_x, lig_xb, lrow_sp, lea_s, lrbf_s,
                                   lseg2d, lstarts, lnblk, N_LIG)
    for q in p["pro_convs_a"]:
        pro_x, pro_xb = _gec_layer(q, pro_x, pro_xb, prow_sp, pea_s, prbf_s,
                                   pseg2d, pstarts, pnblk, N_PRO)
    (lig_x, lig_xb), lig_vn = _vn_layer(p["lig_vn_a"], lig_x, lbatch2d,
                                        lstarts_b, lig_vn)
    (pro_x, pro_xb), pro_vn = _vn_layer(p["pro_vn_a"], pro_x, pbatch2d,
                                        pstarts_b, pro_vn)
    (pro_x, pro_xb), (lig_x, lig_xb) = _ca_apply(
        p["ca1"], pro_x, lig_x, pi_sP_p, li_sP_p, cp_seg2d, cp_starts,
        cnblk_p, li_sL_p, pi_sL_p, cl_seg2d, cl_starts, cnblk_l, cea_sP,
        cea_sL)

    for q in p["lig_convs_b"]:
        lig_x, lig_xb = _gec_layer(q, lig_x, lig_xb, lrow_sp, lea_s, lrbf_s,
                                   lseg2d, lstarts, lnblk, N_LIG)
    for q in p["pro_convs_b"]:
        pro_x, pro_xb = _gec_layer(q, pro_x, pro_xb, prow_sp, pea_s, prbf_s,
                                   pseg2d, pstarts, pnblk, N_PRO)
    (lig_x, lig_xb), lig_vn = _vn_layer(p["lig_vn_b"], lig_x, lbatch2d,
                                        lstarts_b, lig_vn)
    (pro_x, pro_xb), pro_vn = _vn_layer(p["pro_vn_b"], pro_x, pbatch2d,
                                        pstarts_b, pro_vn)
    (pro_x, pro_xb), (lig_x, lig_xb) = _ca_apply(
        p["ca2"], pro_x, lig_x, pi_sP_p, li_sP_p, cp_seg2d, cp_starts,
        cnblk_p, li_sL_p, pi_sL_p, cl_seg2d, cl_starts, cnblk_l, cea_sP,
        cea_sL)

    l_attn, l_den, l_sums, l_cnt = _attn_pool(p["lig_pool"], lig_x,
                                              lbatch2d, lstarts_b)
    p_attn, p_den, p_sums, p_cnt = _attn_pool(p["pro_pool"], pro_x,
                                              pbatch2d, pstarts_b)
    W1 = p["head1"]["W"]
    h = _dense(
        [(l_attn, W1[0:H]), (l_sums, W1[H:2 * H]),
         (p_attn, W1[2 * H:3 * H]), (p_sums, W1[3 * H:4 * H]),
         (lig_vn, W1[4 * H:5 * H]), (pro_vn, W1[5 * H:6 * H])],
        p["head1"]["b"], "silu", p["head2"]["W"], p["head2"]["b"], "silu",
        None, None, divs=[(l_den, "eps"), (l_cnt, "cnt"), (p_den, "eps"),
                          (p_cnt, "cnt"), None, None])
    return _dense([(h, p["head3"]["W"])], p["head3"]["b"], None, None, None,
                  None, None, None)


# speedup vs baseline: 1.0330x; 1.0330x over previous
import functools

import jax
import jax.numpy as jnp
from jax import lax
from jax.experimental import pallas as pl
from jax.experimental.pallas import tpu as pltpu
from jax.experimental.pallas import tpu_sc as plsc

F32 = jnp.float32
I32 = jnp.int32

H = 384
B = 256
BLK_N = 256
CSR_CHUNK = 2048
SC_NW = 32
SC_CH = 128
SC_ALIGN = SC_NW * SC_CH
_SENT = 2 ** 30


def _cdiv(a, b):
    return (a + b - 1) // b


def _blk_rows(rows):
    for c in (512, 400, 256, 128, 64, 32, 16, 8):
        if rows % c == 0:
            return c
    raise ValueError(f"bad row count {rows}")


SC_NB = 2


def _sc_gather_multi(specs):
    D = specs[0][0].shape[1]
    dt = specs[0][0].dtype
    nt = len(specs)
    Es = [idx.shape[0] for _, idx in specs]
    for t, i in specs:
        assert t.shape[1] == D and t.dtype == dt
        assert i.shape[0] % SC_ALIGN == 0
    mesh = plsc.VectorSubcoreMesh(core_axis_name="c", subcore_axis_name="s")

    @functools.partial(
        pl.kernel, mesh=mesh,
        compiler_params=pltpu.CompilerParams(
            use_tc_tiling_on_sc=(D % 128 == 0)),
        out_type=[jax.ShapeDtypeStruct((E, D), dt) for E in Es],
        scratch_types=(
            [pltpu.VMEM((SC_CH,), I32) for _ in range(SC_NB)]
            + [pltpu.VMEM((SC_CH, D), dt) for _ in range(SC_NB)]
            + [pltpu.SemaphoreType.DMA] * (3 * SC_NB)
        ),
    )
    def k(*refs):
        tab_refs = refs[:nt]
        idx_refs = refs[nt:2 * nt]
        out_refs = refs[2 * nt:3 * nt]
        sc = refs[3 * nt:]
        idx_bufs = sc[:SC_NB]
        row_bufs = sc[SC_NB:2 * SC_NB]
        sem_i = sc[2 * SC_NB:3 * SC_NB]
        sem_g = sc[3 * SC_NB:4 * SC_NB]
        sem_o = sc[4 * SC_NB:5 * SC_NB]
        wid = lax.axis_index("s") * 2 + lax.axis_index("c")

        def run_batch(t, gbase, nb):
            base = [gbase + b * SC_CH for b in range(nb)]
            hi = [pltpu.async_copy(
                idx_refs[t].at[pl.ds(base[b], SC_CH)], idx_bufs[b], sem_i[b])
                for b in range(nb)]
            for b in range(nb):
                hi[b].wait()
            hg = [pltpu.async_copy(
                tab_refs[t].at[idx_bufs[b]], row_bufs[b], sem_g[b])
                for b in range(nb)]
            for b in range(nb):
                hg[b].wait()
            ho = [pltpu.async_copy(
                row_bufs[b], out_refs[t].at[pl.ds(base[b], SC_CH)], sem_o[b])
                for b in range(nb)]
            for b in range(nb):
                ho[b].wait()

        for t in range(nt):
            per_w = Es[t] // SC_NW
            n_ch = per_w // SC_CH
            n_grp, rem = n_ch // SC_NB, n_ch % SC_NB

            if n_grp:
                def grp_body(g, carry, t=t, per_w=per_w):
                    run_batch(t, wid * per_w + g * (SC_NB * SC_CH), SC_NB)
                    return carry
                lax.fori_loop(0, n_grp, grp_body, 0)
            if rem:
                run_batch(t, wid * per_w + n_grp * (SC_NB * SC_CH), rem)

    outs = k(*[t for t, _ in specs], *[i for _, i in specs])
    return outs if isinstance(outs, (list, tuple)) else [outs]


def _sc_gather(table, idx):
    return _sc_gather_multi([(table, idx)])[0]


def _csr_segsum(vals, seg2d, starts, nblk, D, aux16=None, scale_by_aux=False,
                also_unscaled=False, with_cnt=False, chunk=CSR_CHUNK):
    rows = vals.shape[0]
    chunk = min(chunk, rows)
    assert rows % 8 == 0
    n_out = 1 + (aux16 is not None) + also_unscaled + with_cnt

    def body(*refs):
        starts_ref, vals_ref = refs[0], refs[1]
        k = 2
        aux_ref = None
        if aux16 is not None:
            aux_ref = refs[k]
            k += 1
        seg_ref = refs[k]
        k += 1
        outs = refs[k:k + n_out]
        k += n_out
        vchunk, schunk, sem_v, sem_s = refs[k:k + 4]
        k += 4
        achunk = sem_a = None
        if aux16 is not None:
            achunk, sem_a = refs[k:k + 2]

        i = pl.program_id(0)
        start = starts_ref[i]
        end = starts_ref[i + 1]
        for o_ref in outs:
            o_ref[...] = jnp.zeros_like(o_ref)
        start8 = (start // 8) * 8
        nch = (end - start8 + chunk - 1) // chunk

        def chunk_body(c, carry):
            cb = start8 + c * chunk
            o = jnp.minimum(cb, rows - chunk)
            cps = [pltpu.make_async_copy(
                vals_ref.at[pl.ds(o, chunk)], vchunk, sem_v),
                pltpu.make_async_copy(
                seg_ref.at[pl.ds(o, chunk)], schunk, sem_s)]
            if aux_ref is not None:
                cps.append(pltpu.make_async_copy(
                    aux_ref.at[pl.ds(o, chunk)], achunk, sem_a))
            for cp in cps:
                cp.start()
            for cp in cps:
                cp.wait()
            g = o + lax.broadcasted_iota(I32, (chunk, 1), 0)
            valid = ((g >= jnp.maximum(start, cb))
                     & (g < jnp.minimum(end, cb + chunk)))
            local = schunk[...] - i * BLK_N
            oh = ((local == lax.broadcasted_iota(I32, (chunk, BLK_N), 1))
                  & valid).astype(F32)

            def acc(o_ref, rhs):
                o_ref[...] += lax.dot_general(
                    oh, rhs, (((0,), (0,)), ((), ())),
                    precision=lax.Precision.HIGHEST,
                    preferred_element_type=F32)

            oi = 0
            v = vchunk[...]
            if scale_by_aux:
                acc(outs[oi], achunk[...][:, :1] * v)
            else:
                acc(outs[oi], v)
            oi += 1
            if aux16 is not None:
                acc(outs[oi], achunk[...])
                oi += 1
            if also_unscaled:
                acc(outs[oi], v)
                oi += 1
            if with_cnt:
                acc(outs[oi], jnp.ones((chunk, 16), F32))
            return carry

        lax.fori_loop(0, nch, chunk_body, 0)

    in_specs = [pl.BlockSpec(memory_space=pltpu.SMEM),
                pl.BlockSpec(memory_space=pl.ANY)]
    args = [starts, vals]
    if aux16 is not None:
        in_specs.append(pl.BlockSpec(memory_space=pl.ANY))
        args.append(aux16)
    in_specs.append(pl.BlockSpec(memory_space=pl.ANY))
    args.append(seg2d)

    out_D = ([D] + ([16] if aux16 is not None else [])
             + ([D] if also_unscaled else []) + ([16] if with_cnt else []))
    scratch = [pltpu.VMEM((chunk, D), F32),
               pltpu.VMEM((chunk, 1), I32),
               pltpu.SemaphoreType.DMA,
               pltpu.SemaphoreType.DMA]
    if aux16 is not None:
        scratch += [pltpu.VMEM((chunk, 16), F32), pltpu.SemaphoreType.DMA]

    res = pl.pallas_call(
        body,
        grid=(nblk,),
        in_specs=in_specs,
        out_specs=[pl.BlockSpec((BLK_N, d), lambda i: (i, 0)) for d in out_D],
        out_shape=[jax.ShapeDtypeStruct((nblk * BLK_N, d), F32) for d in out_D],
        scratch_shapes=scratch,
    )(*args)
    return res[0] if n_out == 1 else res


_BF16_GATHER = False


def _dense(parts, b1, mid, W2, b2, out_act, res, ln, divs=None,
           bf16_copy=False):
    dup_out = bf16_copy and not _BF16_GATHER
    bf16_copy = bf16_copy and _BF16_GATHER
    rows = parts[0][0].shape[0]
    blk = _blk_rows(rows)
    grid = rows // blk
    H1 = parts[0][1].shape[1]
    Hout = W2.shape[1] if W2 is not None else H1
    divs = divs or [None] * len(parts)

    arrs = [a for a, _ in parts]
    Ws = [jnp.asarray(w) for _, w in parts]
    n = len(parts)
    div_idx = [j for j, d in enumerate(divs) if d is not None]
    div_arrs = [divs[j][0] for j in div_idx]
    div_modes = [divs[j][1] for j in div_idx]

    tail = [b1.reshape(1, -1)]
    if W2 is not None:
        tail += [W2, b2.reshape(1, -1)]
    if ln is not None:
        tail += [ln["g"].reshape(1, -1), ln["b"].reshape(1, -1)]
    if res is not None:
        tail += [res]

    def body(*refs):
        a_refs = refs[:n]
        w_refs = refs[n:2 * n]
        pos = 2 * n
        d_refs = refs[pos:pos + len(div_arrs)]
        pos += len(div_arrs)
        b1_ref = refs[pos]
        pos += 1
        if W2 is not None:
            W2_ref, b2_ref = refs[pos], refs[pos + 1]
            pos += 2
        if ln is not None:
            g_ref, be_ref = refs[pos], refs[pos + 1]
            pos += 2
        if res is not None:
            r_ref = refs[pos]
            pos += 1
        out_ref = refs[pos:pos + 2] if bf16_copy else refs[pos]

        h = None
        di = 0
        for j in range(n):
            a = a_refs[j][...]
            if j in div_idx:
                dv = d_refs[di][...][:, :1]
                if div_modes[di] == "cnt":
                    a = a / jnp.maximum(dv, 1.0)
                else:
                    a = a / (dv + 1e-8)
                di += 1
            t = jnp.dot(a, w_refs[j][...], preferred_element_type=F32)
            h = t if h is None else h + t
        h = h + b1_ref[...]
        if mid == "silu":
            h = h * jax.nn.sigmoid(h)
        elif mid == "tanh":
            h = jnp.tanh(h)
        if W2 is not None:
            h = jnp.dot(h, W2_ref[...], preferred_element_type=F32) + b2_ref[...]
        if out_act == "silu":
            h = h * jax.nn.sigmoid(h)
        if res is not None:
            h = h + r_ref[...]
        if ln is not None:
            m = jnp.mean(h, axis=1, keepdims=True)
            v = jnp.mean((h - m) ** 2, axis=1, keepdims=True)
            h = (h - m) * lax.rsqrt(v + 1e-5) * g_ref[...] + be_ref[...]
        if bf16_copy:
            out_ref, bf_ref = out_ref
            bf_ref[...] = h.astype(jnp.bfloat16)
        out_ref[...] = h

    in_specs = (
        [pl.BlockSpec((blk, a.shape[1]), lambda i: (i, 0)) for a in arrs]
        + [pl.BlockSpec(w.shape, lambda i: (0, 0)) for w in Ws]
        + [pl.BlockSpec((blk, d.shape[1]), lambda i: (i, 0)) for d in div_arrs]
    )
    for t in tail[:-1] if res is not None else tail:
        in_specs.append(pl.BlockSpec(t.shape, lambda i: (0, 0)))
    if res is not None:
        in_specs.append(pl.BlockSpec((blk, Hout), lambda i: (i, 0)))

    res_ = pl.pallas_call(
        body,
        grid=(grid,),
        in_specs=in_specs,
        out_specs=([pl.BlockSpec((blk, Hout), lambda i: (i, 0))] * 2
                   if bf16_copy else
                   pl.BlockSpec((blk, Hout), lambda i: (i, 0))),
        out_shape=([jax.ShapeDtypeStruct((rows, Hout), F32),
                    jax.ShapeDtypeStruct((rows, Hout), jnp.bfloat16)]
                   if bf16_copy else
                   jax.ShapeDtypeStruct((rows, Hout), F32)),
    )(*arrs, *Ws, *div_arrs, *tail)
    if dup_out:
        return res_, res_
    return res_


def _gec_edge(ea_s, rbf_s, xg, q):
    rows = ea_s.shape[0]
    blk = 512
    edp = ea_s.shape[1]
    W1a = _zero_pad_W(q["ee1"]["W"][:-16], edp)
    W1b = q["ee1"]["W"][-16:]
    Wm = q["mm1"]["W"]

    def body(ea_ref, rb_ref, xg_ref, w1a, w1b, b1, w2, b2, wma, wmb, bm,
             wm2, bm2, out):
        h = (jnp.dot(ea_ref[...], w1a[...], preferred_element_type=F32)
             + jnp.dot(rb_ref[...], w1b[...], preferred_element_type=F32)
             + b1[...])
        h = h * jax.nn.sigmoid(h)
        e = jnp.dot(h, w2[...], preferred_element_type=F32) + b2[...]
        m = (jnp.dot(xg_ref[...].astype(F32), wma[...],
                     preferred_element_type=F32)
             + jnp.dot(e, wmb[...], preferred_element_type=F32) + bm[...])
        m = m * jax.nn.sigmoid(m)
        out[...] = jnp.dot(m, wm2[...], preferred_element_type=F32) + bm2[...]

    args = [ea_s, rbf_s, xg, W1a, W1b, q["ee1"]["b"].reshape(1, -1),
            q["ee2"]["W"], q["ee2"]["b"].reshape(1, -1),
            Wm[:H], Wm[H:], q["mm1"]["b"].reshape(1, -1),
            q["mm2"]["W"], q["mm2"]["b"].reshape(1, -1)]
    in_specs = [pl.BlockSpec((blk, edp), lambda i: (i, 0)),
                pl.BlockSpec((blk, 16), lambda i: (i, 0)),
                pl.BlockSpec((blk, H), lambda i: (i, 0))]
    in_specs += [pl.BlockSpec(a.shape, lambda i: (0, 0)) for a in args[3:]]
    return pl.pallas_call(
        body, grid=(rows // blk,),
        in_specs=in_specs,
        out_specs=pl.BlockSpec((blk, H), lambda i: (i, 0)),
        out_shape=jax.ShapeDtypeStruct((rows, H), F32),
    )(*args)


def _rbf16(posr, posc):
    rows = posr.shape[0]
    blk = _blk_rows(rows)

    def body(pr, pc, out):
        diff = pr[...] - pc[...]
        d = jnp.sqrt(jnp.sum(diff * diff, axis=1, keepdims=True))
        c = lax.broadcasted_iota(I32, (1, 16), 1).astype(F32) * (10.0 / 15.0)
        out[...] = jnp.exp(-((d - c) ** 2) / (2.0 * 0.625 ** 2))

    return pl.pallas_call(
        body, grid=(rows // blk,),
        in_specs=[pl.BlockSpec((blk, 16), lambda i: (i, 0))] * 2,
        out_specs=pl.BlockSpec((blk, 16), lambda i: (i, 0)),
        out_shape=jax.ShapeDtypeStruct((rows, 16), F32),
    )(posr, posc)


def _score16(Qg, Kg, ceag, Web):
    rows = Qg.shape[0]
    blk = _blk_rows(rows)
    scale = float(H) ** -0.5

    def body(q, k, ce, w, out):
        s = jnp.sum(q[...] * k[...], axis=1, keepdims=True) * scale
        s = s + jnp.dot(ce[...], w[...], preferred_element_type=F32)
        out[...] = jnp.broadcast_to(s, (blk, 16))

    return pl.pallas_call(
        body, grid=(rows // blk,),
        in_specs=[
            pl.BlockSpec((blk, H), lambda i: (i, 0)),
            pl.BlockSpec((blk, H), lambda i: (i, 0)),
            pl.BlockSpec((blk, ceag.shape[1]), lambda i: (i, 0)),
            pl.BlockSpec(Web.shape, lambda i: (0, 0)),
        ],
        out_specs=pl.BlockSpec((blk, 16), lambda i: (i, 0)),
        out_shape=jax.ShapeDtypeStruct((rows, 16), F32),
    )(Qg, Kg, ceag, Web)


def _gmax(x):
    rows, C = x.shape
    blk = _blk_rows(rows)

    def body(x_ref, out):
        i = pl.program_id(0)
        m = jnp.max(x_ref[...]).reshape(1, 1)
        prev = jnp.where(i == 0, m, out[...])
        out[...] = jnp.maximum(prev, m)

    return pl.pallas_call(
        body, grid=(rows // blk,),
        in_specs=[pl.BlockSpec((blk, C), lambda i: (i, 0))],
        out_specs=pl.BlockSpec((1, 1), lambda i: (0, 0)),
        out_shape=jax.ShapeDtypeStruct((1, 1), F32),
    )(x)


def _sub_exp16(x, m):
    rows, C = x.shape
    blk = _blk_rows(rows)

    def body(x_ref, m_ref, out):
        v = jnp.exp(x_ref[...] - m_ref[...])
        out[...] = jnp.broadcast_to(v[:, :1], (blk, 16))

    return pl.pallas_call(
        body, grid=(rows // blk,),
        in_specs=[pl.BlockSpec((blk, C), lambda i: (i, 0)),
                  pl.BlockSpec((1, 1), lambda i: (0, 0))],
        out_specs=pl.BlockSpec((blk, 16), lambda i: (i, 0)),
        out_shape=jax.ShapeDtypeStruct((rows, 16), F32),
    )(x, m)


def _emul(e16, V):
    rows, D = V.shape
    blk = _blk_rows(rows)

    def body(e_ref, v_ref, out):
        out[...] = e_ref[...][:, :1] * v_ref[...]

    return pl.pallas_call(
        body, grid=(rows // blk,),
        in_specs=[pl.BlockSpec((blk, 16), lambda i: (i, 0)),
                  pl.BlockSpec((blk, D), lambda i: (i, 0))],
        out_specs=pl.BlockSpec((blk, D), lambda i: (i, 0)),
        out_shape=jax.ShapeDtypeStruct((rows, D), F32),
    )(e16, V)


def _bcast256(batch2d, table):
    rows = batch2d.shape[0]
    D = table.shape[1]
    blk = _blk_rows(rows)

    def body(b_ref, t_ref, out):
        oh = (b_ref[...] == lax.broadcasted_iota(I32, (blk, 256), 1)
              ).astype(F32)
        out[...] = lax.dot_general(
            oh, t_ref[...], (((1,), (0,)), ((), ())),
            precision=lax.Precision.HIGHEST, preferred_element_type=F32)

    return pl.pallas_call(
        body, grid=(rows // blk,),
        in_specs=[pl.BlockSpec((blk, 1), lambda i: (i, 0)),
                  pl.BlockSpec((256, D), lambda i: (0, 0))],
        out_specs=pl.BlockSpec((blk, D), lambda i: (i, 0)),
        out_shape=jax.ShapeDtypeStruct((rows, D), F32),
    )(batch2d, table)


def _pad_cols(x, width):
    t = width - x.shape[1]
    return jnp.pad(x, ((0, 0), (0, t))) if t else x


def _pad_idx(idx, mult):
    t = (-idx.shape[0]) % mult
    return jnp.pad(idx, (0, t)) if t else idx


def _seg2d(seg_sorted, rows):
    t = rows - seg_sorted.shape[0]
    s = (jnp.pad(seg_sorted, (0, t), constant_values=_SENT)
         if t else seg_sorted)
    return s.reshape(rows, 1)


def _starts(seg_sorted, nblk):
    bounds = jnp.arange(nblk + 1, dtype=I32) * BLK_N
    return jnp.searchsorted(seg_sorted, bounds).astype(I32)


def _zero_pad_W(Wpart, width):
    t = width - Wpart.shape[0]
    return jnp.pad(Wpart, ((0, t), (0, 0))) if t else Wpart


def _pack_i32(xb):
    if xb.dtype == F32:
        return xb
    r, c = xb.shape
    return lax.bitcast_convert_type(xb.reshape(r, c // 2, 2), I32)


def _unpack_bf16(xi):
    if xi.dtype == F32:
        return xi
    r, c = xi.shape
    return lax.bitcast_convert_type(xi, jnp.bfloat16).reshape(r, 2 * c)


def _gec_layer(q, x, x_bf, xg_idx, ea_s, rbf_s, seg2d, starts, nblk, N):
    xg = _unpack_bf16(_sc_gather(_pack_i32(x_bf), xg_idx))
    msg = _gec_edge(ea_s, rbf_s, xg, q)
    agg = _csr_segsum(msg, seg2d, starts, nblk, H)[:N]
    Wu = q["um1"]["W"]
    return _dense(
        [(x, Wu[:H]), (agg, Wu[H:])],
        q["um1"]["b"], "silu", q["um2"]["W"], q["um2"]["b"], None, x,
        q["norm"], bf16_copy=True)


def _vn_layer(q, x, batch2d, starts_b, vn):
    sums, cnt = _csr_segsum(x, batch2d, starts_b, 1, H, with_cnt=True,
                            chunk=4096)
    Wt = q["to1"]["W"]
    vn_new = _dense(
        [(sums, Wt[:H]), (vn, Wt[H:])],
        q["to1"]["b"], "silu", q["to2"]["W"], q["to2"]["b"], None, vn,
        q["vn_norm"], divs=[(cnt, "cnt"), None])
    vng = _bcast256(batch2d, vn_new)
    Wf = q["fr1"]["W"]
    x_new = _dense(
        [(x, Wf[:H]), (vng, Wf[H:])],
        q["fr1"]["b"], "silu", q["fr2"]["W"], q["fr2"]["b"], None, x,
        q["node_norm"], bf16_copy=True)
    return x_new, vn_new


def _ca_side(x_tgt, Qg, Kg, Vg, cea_s, Web, seg2d, starts, nblk, Wout, bout,
             norm):
    N = x_tgt.shape[0]
    s16 = _score16(Qg, Kg, cea_s, Web)
    smax = _gmax(s16)
    e16 = _sub_exp16(s16, smax)
    upd_raw, den16 = _csr_segsum(Vg, seg2d, starts, nblk, H, aux16=e16,
                                 scale_by_aux=True)
    return _dense([(upd_raw[:N], Wout)], bout, None, None, None, None, x_tgt,
                  norm, divs=[(den16[:N], "eps")], bf16_copy=True)


def _ca_apply(q, pro_x, lig_x, pi_sp, li_sp, pi_seg2d, pstarts, nblk_p,
              li_sp2, pi_sp2, li_seg2d, lstarts, nblk_l, cea_sp, cea_sl):
    Web = _zero_pad_W(q["eb"]["W"], 32)
    lin = lambda x, w: _dense([(x, w["W"])], w["b"], None, None, None, None,
                              None, None)
    q_pro, k_lig, v_lig = lin(pro_x, q["q_pro"]), lin(lig_x, q["k_lig"]), lin(lig_x, q["v_lig"])
    q_lig, k_pro, v_pro = lin(lig_x, q["q_lig"]), lin(pro_x, q["k_pro"]), lin(pro_x, q["v_pro"])
    QgP, KgP, VgP, QgL, KgL, VgL = _sc_gather_multi([
        (q_pro, pi_sp), (k_lig, li_sp), (v_lig, li_sp),
        (q_lig, li_sp2), (k_pro, pi_sp2), (v_pro, pi_sp2)])
    pro_new = _ca_side(pro_x, QgP, KgP, VgP, cea_sp, Web, pi_seg2d, pstarts,
                       nblk_p, q["out_pro"]["W"], q["out_pro"]["b"],
                       q["norm_pro"])
    lig_new = _ca_side(lig_x, QgL, KgL, VgL, cea_sl, Web, li_seg2d, lstarts,
                       nblk_l, q["out_lig"]["W"], q["out_lig"]["b"],
                       q["norm_lig"])
    return pro_new, lig_new


def _attn_pool(q, x, batch2d, starts_b):
    g = _dense([(x, q["g1"]["W"])], q["g1"]["b"], "tanh", q["g2"]["W"],
               q["g2"]["b"], None, None, None)
    gmax = _gmax(g)
    e16 = _sub_exp16(g, gmax)
    attn_raw, den16, sums, cnt = _csr_segsum(
        x, batch2d, starts_b, 1, H, aux16=e16, scale_by_aux=True,
        also_unscaled=True, with_cnt=True, chunk=4096)
    return attn_raw, den16, sums, cnt


def kernel(ligand_x, ligand_edge_index, ligand_edge_attr, ligand_pos,
           ligand_batch, protein_x, protein_edge_index, protein_edge_attr,
           protein_pos, protein_batch, cross_edge_index, cross_edge_attr,
           params):
    p = params
    N_LIG, N_PRO = ligand_x.shape[0], protein_x.shape[0]
    E_LIG = ligand_edge_index.shape[1]
    E_PRO = protein_edge_index.shape[1]
    E_CROSS = cross_edge_index.shape[1]

    def edge_prep(ei, E, N):
        row, col = ei[0], ei[1]
        perm = jnp.argsort(col)
        row_s, col_s = row[perm], col[perm]
        E_pad = E + ((-E) % SC_ALIGN)
        nblk = _cdiv(N, BLK_N)
        return (_pad_idx(perm, SC_ALIGN), _pad_idx(row_s, SC_ALIGN),
                _pad_idx(col_s, SC_ALIGN), _seg2d(col_s, E_pad),
                _starts(col_s, nblk), nblk)

    (lperm_p, lrow_sp, lcol_sp, lseg2d, lstarts, lnblk) = edge_prep(
        ligand_edge_index, E_LIG, N_LIG)
    (pperm_p, prow_sp, pcol_sp, pseg2d, pstarts, pnblk) = edge_prep(
        protein_edge_index, E_PRO, N_PRO)

    ci, cj = cross_edge_index[0], cross_edge_index[1]
    cpermP = jnp.argsort(ci)
    pi_sP, li_sP = ci[cpermP], cj[cpermP]
    cpermL = jnp.argsort(cj)
    pi_sL, li_sL = ci[cpermL], cj[cpermL]
    E_CR_P = E_CROSS + ((-E_CROSS) % SC_ALIGN)
    cnblk_p, cnblk_l = _cdiv(N_PRO, BLK_N), _cdiv(N_LIG, BLK_N)
    cp_seg2d, cp_starts = _seg2d(pi_sP, E_CR_P), _starts(pi_sP, cnblk_p)
    cl_seg2d, cl_starts = _seg2d(li_sL, E_CR_P), _starts(li_sL, cnblk_l)
    pi_sP_p, li_sP_p = _pad_idx(pi_sP, SC_ALIGN), _pad_idx(li_sP, SC_ALIGN)
    pi_sL_p, li_sL_p = _pad_idx(pi_sL, SC_ALIGN), _pad_idx(li_sL, SC_ALIGN)

    lbatch2d = ligand_batch.reshape(N_LIG, 1)
    pbatch2d = protein_batch.reshape(N_PRO, 1)
    lstarts_b = jnp.array([0, N_LIG], I32)
    pstarts_b = jnp.array([0, N_PRO], I32)

    lpos16 = _pad_cols(ligand_pos, 16)
    ppos16 = _pad_cols(protein_pos, 16)
    lposr, lposc, pposr, pposc = _sc_gather_multi([
        (lpos16, lrow_sp), (lpos16, lcol_sp),
        (ppos16, prow_sp), (ppos16, pcol_sp)])
    lrbf_s = _rbf16(lposr, lposc)
    prbf_s = _rbf16(pposr, pposc)
    cea32 = _pad_cols(cross_edge_attr, 32)
    lea_s, pea_s, cea_sP, cea_sL = _sc_gather_multi([
        (_pad_cols(ligand_edge_attr, 32), lperm_p),
        (_pad_cols(protein_edge_attr, 32), pperm_p),
        (cea32, _pad_idx(cpermP, SC_ALIGN)),
        (cea32, _pad_idx(cpermL, SC_ALIGN))])

    lig_x, lig_xb = _dense([(ligand_x, p["lig_in"]["W"])], p["lig_in"]["b"],
                           "silu", None, None, None, None, None,
                           bf16_copy=True)
    pro_x, pro_xb = _dense([(protein_x, p["pro_in"]["W"])], p["pro_in"]["b"],
                           "silu", None, None, None, None, None,
                           bf16_copy=True)
    lig_vn = jnp.tile(p["lig_vn_init"], (B, 1))
    pro_vn = jnp.tile(p["pro_vn_init"], (B, 1))

    for q in p["lig_convs_a"]:
        lig_x, lig_xb = _gec_layer(q, lig_x, lig_xb, lrow_sp, lea_s, lrbf_s,
                                   lseg2d, lstarts, lnblk, N_LIG)
    for q in p["pro_convs_a"]:
        pro_x, pro_xb = _gec_layer(q, pro_x, pro_xb, prow_sp, pea_s, prbf_s,
                                   pseg2d, pstarts, pnblk, N_PRO)
    (lig_x, lig_xb), lig_vn = _vn_layer(p["lig_vn_a"], lig_x, lbatch2d,
                                        lstarts_b, lig_vn)
    (pro_x, pro_xb), pro_vn = _vn_layer(p["pro_vn_a"], pro_x, pbatch2d,
                                        pstarts_b, pro_vn)
    (pro_x, pro_xb), (lig_x, lig_xb) = _ca_apply(
        p["ca1"], pro_x, lig_x, pi_sP_p, li_sP_p, cp_seg2d, cp_starts,
        cnblk_p, li_sL_p, pi_sL_p, cl_seg2d, cl_starts, cnblk_l, cea_sP,
        cea_sL)

    for q in p["lig_convs_b"]:
        lig_x, lig_xb = _gec_layer(q, lig_x, lig_xb, lrow_sp, lea_s, lrbf_s,
                                   lseg2d, lstarts, lnblk, N_LIG)
    for q in p["pro_convs_b"]:
        pro_x, pro_xb = _gec_layer(q, pro_x, pro_xb, prow_sp, pea_s, prbf_s,
                                   pseg2d, pstarts, pnblk, N_PRO)
    (lig_x, lig_xb), lig_vn = _vn_layer(p["lig_vn_b"], lig_x, lbatch2d,
                                        lstarts_b, lig_vn)
    (pro_x, pro_xb), pro_vn = _vn_layer(p["pro_vn_b"], pro_x, pbatch2d,
                                        pstarts_b, pro_vn)
    (pro_x, pro_xb), (lig_x, lig_xb) = _ca_apply(
        p["ca2"], pro_x, lig_x, pi_sP_p, li_sP_p, cp_seg2d, cp_starts,
        cnblk_p, li_sL_p, pi_sL_p, cl_seg2d, cl_starts, cnblk_l, cea_sP,
        cea_sL)

    l_attn, l_den, l_sums, l_cnt = _attn_pool(p["lig_pool"], lig_x,
                                              lbatch2d, lstarts_b)
    p_attn, p_den, p_sums, p_cnt = _attn_pool(p["pro_pool"], pro_x,
                                              pbatch2d, pstarts_b)
    W1 = p["head1"]["W"]
    h = _dense(
        [(l_attn, W1[0:H]), (l_sums, W1[H:2 * H]),
         (p_attn, W1[2 * H:3 * H]), (p_sums, W1[3 * H:4 * H]),
         (lig_vn, W1[4 * H:5 * H]), (pro_vn, W1[5 * H:6 * H])],
        p["head1"]["b"], "silu", p["head2"]["W"], p["head2"]["b"], "silu",
        None, None, divs=[(l_den, "eps"), (l_cnt, "cnt"), (p_den, "eps"),
                          (p_cnt, "cnt"), None, None])
    return _dense([(h, p["head3"]["W"])], p["head3"]["b"], None, None, None,
                  None, None, None)
